# baseline (speedup 1.0000x reference)
import functools

import jax
import jax.numpy as jnp
from jax import lax
from jax.experimental import pallas as pl
from jax.experimental.pallas import tpu as pltpu
from jax.experimental.pallas import tpu_sc as plsc

N_NODES = 10000
D_FEAT = 128
MASK_NUM = 2000

BLK = 1000
NB = N_NODES // BLK


def _copy_mean_body(x_ref, out_ref, tok_ref, acc_ref):
    i = pl.program_id(0)
    xv = x_ref[...]
    out_ref[...] = xv
    partial = jnp.sum(xv, axis=0, keepdims=True)

    @pl.when(i == 0)
    def _():
        acc_ref[...] = partial

    @pl.when(i > 0)
    def _():
        acc_ref[...] = acc_ref[...] + partial

    @pl.when(i == NB - 1)
    def _():
        tok_ref[...] = acc_ref[...] * (1.0 / N_NODES)


_copy_mean = pl.pallas_call(
    _copy_mean_body,
    grid=(NB,),
    in_specs=[pl.BlockSpec((BLK, D_FEAT), lambda i: (i, 0))],
    out_specs=(
        pl.BlockSpec((BLK, D_FEAT), lambda i: (i, 0)),
        pl.BlockSpec((1, D_FEAT), lambda i: (0, 0)),
    ),
    out_shape=(
        jax.ShapeDtypeStruct((N_NODES, D_FEAT), jnp.float32),
        jax.ShapeDtypeStruct((1, D_FEAT), jnp.float32),
    ),
    scratch_shapes=[pltpu.VMEM((1, D_FEAT), jnp.float32)],
)

NC = 2
NS = 16
NW = NC * NS
IDX_PAD = 2048
PER_W = IDX_PAD // NW

_sc_mesh = plsc.VectorSubcoreMesh(core_axis_name="c", subcore_axis_name="s")


@functools.partial(
    pl.kernel,
    mesh=_sc_mesh,
    scratch_types=[
        pltpu.VMEM((PER_W,), jnp.int32),
        pltpu.VMEM((1, D_FEAT), jnp.float32),
        pltpu.VMEM((PER_W, D_FEAT), jnp.float32),
        pltpu.SemaphoreType.DMA,
    ],
)
def _sc_scatter(out_ref, tok_hbm, idx_hbm, idx_v, tok_v, rows_v, sem):
    wid = lax.axis_index("s") * NC + lax.axis_index("c")
    base = wid * PER_W
    pltpu.sync_copy(idx_hbm.at[pl.ds(base, PER_W)], idx_v)
    pltpu.sync_copy(tok_hbm, tok_v)
    for r in range(PER_W):
        for c in range(D_FEAT // 16):
            rows_v[r, pl.ds(c * 16, 16)] = tok_v[0, pl.ds(c * 16, 16)]
    pltpu.async_copy(rows_v, out_ref.at[idx_v], sem).wait()


def kernel(x, edge_index, mask_idx):
    idx = mask_idx.astype(jnp.int32)
    idx_padded = jnp.concatenate(
        [idx, jnp.broadcast_to(idx[:1], (IDX_PAD - MASK_NUM,))]
    )
    x_copy, token = _copy_mean(x)
    x_ref = jax.new_ref(x_copy)
    _sc_scatter(x_ref, token, idx_padded)
    x_new = jax.freeze(x_ref)
    return (x_new, edge_index)


# speedup vs baseline: 3.5385x; 3.5385x over previous
import jax
import jax.numpy as jnp
from jax.experimental import pallas as pl
from jax.experimental.pallas import tpu as pltpu

N_NODES = 10000
D_FEAT = 128
MASK_NUM = 2000

N_CHUNK = 4
CHUNK = N_NODES // N_CHUNK


def _attrmask_body(idx_ref, x_ref, out_ref, buf_ref, mask_ref, sem_in, sem_out):
    cps_in = [
        pltpu.make_async_copy(
            x_ref.at[pl.ds(k * CHUNK, CHUNK), :],
            buf_ref.at[pl.ds(k * CHUNK, CHUNK), :],
            sem_in,
        )
        for k in range(N_CHUNK)
    ]
    for cp in cps_in:
        cp.start()

    mask_ref[...] = jnp.zeros((N_NODES, 1), jnp.float32)
    one = jnp.ones((1, 1), jnp.float32)

    def mbody(i, o):
        mask_ref[pl.ds(idx_ref[i], 1), :] = o
        return o

    jax.lax.fori_loop(0, MASK_NUM, mbody, one, unroll=16)

    acc = jnp.zeros((1, D_FEAT), jnp.float32)
    for k in range(N_CHUNK):
        cps_in[k].wait()
        acc = acc + jnp.sum(
            buf_ref[pl.ds(k * CHUNK, CHUNK), :], axis=0, keepdims=True
        )
    token = acc * (1.0 / N_NODES)

    cps_out = [
        pltpu.make_async_copy(
            buf_ref.at[pl.ds(k * CHUNK, CHUNK), :],
            out_ref.at[pl.ds(k * CHUNK, CHUNK), :],
            sem_out,
        )
        for k in range(N_CHUNK)
    ]
    for k in range(N_CHUNK):
        sl = pl.ds(k * CHUNK, CHUNK)
        m = mask_ref[sl, :]
        buf_ref[sl, :] = jnp.where(m > 0.0, token, buf_ref[sl, :])
        cps_out[k].start()
    for k in range(N_CHUNK):
        cps_out[k].wait()


def kernel(x, edge_index, mask_idx):
    idx = mask_idx.astype(jnp.int32)
    x_new = pl.pallas_call(
        _attrmask_body,
        out_shape=jax.ShapeDtypeStruct(x.shape, x.dtype),
        in_specs=[
            pl.BlockSpec(memory_space=pltpu.SMEM),
            pl.BlockSpec(memory_space=pltpu.HBM),
        ],
        out_specs=pl.BlockSpec(memory_space=pltpu.HBM),
        scratch_shapes=[
            pltpu.VMEM((N_NODES, D_FEAT), jnp.float32),
            pltpu.VMEM((N_NODES, 1), jnp.float32),
            pltpu.SemaphoreType.DMA,
            pltpu.SemaphoreType.DMA,
        ],
    )(idx, x)
    return (x_new, edge_index)
